# initial kernel scaffold (unmeasured)
import jax
import jax.numpy as jnp
from jax import lax
from jax.experimental import pallas as pl
from jax.experimental.pallas import tpu as pltpu

B = 16
H = 16
D = 64
SCALE = D ** -0.5
CW = D + 2


def kernel(Q, K, V):
    kv_local = K.shape[1]

    def body(q_ref, k_ref, v_ref, out_ref, send_ref, recv_ref,
             send_sem, recv_sem):
        b = pl.program_id(0)
        nb = pl.num_programs(0)

        for h in range(H):
            k2 = k_ref[0, :, h, :]
            v2 = v_ref[0, :, h, :]
            q2 = q_ref[b, :, h, :]
            s = lax.dot_general(
                q2, k2, (((1,), (1,)), ((), ())),
                preferred_element_type=jnp.float32,
            ) * SCALE
            m = jnp.max(s, axis=1, keepdims=True)
            p = jnp.exp(s - m)
            l = jnp.sum(p, axis=1, keepdims=True)
            o = lax.dot_general(
                p, v2, (((1,), (0,)), ((), ())),
                preferred_element_type=jnp.float32,
            )
            send_ref[b, pl.ds(h, 1), pl.ds(0, D)] = o
            send_ref[b, pl.ds(h, 1), pl.ds(D, 1)] = m
            send_ref[b, pl.ds(h, 1), pl.ds(D + 1, 1)] = l

        @pl.when(b == nb - 1)
        def _():
            my_x = lax.axis_index("x")
            my_y = lax.axis_index("y")
            my_z = lax.axis_index("z")
            nbr = (1 - my_x, my_y, my_z)

            barrier = pltpu.get_barrier_semaphore()
            pl.semaphore_signal(
                barrier, inc=1, device_id=nbr,
                device_id_type=pl.DeviceIdType.MESH,
            )
            pl.semaphore_wait(barrier, 1)

            rdma = pltpu.make_async_remote_copy(
                src_ref=send_ref,
                dst_ref=recv_ref,
                send_sem=send_sem,
                recv_sem=recv_sem,
                device_id=nbr,
                device_id_type=pl.DeviceIdType.MESH,
            )
            rdma.start()
            rdma.wait()

            o_a = send_ref[:, :, 0:D]
            m_a = send_ref[:, :, D:D + 1]
            l_a = send_ref[:, :, D + 1:D + 2]
            o_b = recv_ref[:, :, 0:D]
            m_b = recv_ref[:, :, D:D + 1]
            l_b = recv_ref[:, :, D + 1:D + 2]
            m_n = jnp.maximum(m_a, m_b)
            alpha = jnp.exp(m_a - m_n)
            beta = jnp.exp(m_b - m_n)
            l_n = l_a * alpha + l_b * beta
            out_ref[:, 0, :, :] = (o_a * alpha + o_b * beta) / l_n

    return pl.pallas_call(
        body,
        grid=(B,),
        in_specs=[
            pl.BlockSpec((B, 1, H, D), lambda b: (0, 0, 0, 0)),
            pl.BlockSpec((1, kv_local, H, D), lambda b: (b, 0, 0, 0)),
            pl.BlockSpec((1, kv_local, H, D), lambda b: (b, 0, 0, 0)),
        ],
        out_specs=pl.BlockSpec((B, 1, H, D), lambda b: (0, 0, 0, 0)),
        out_shape=jax.ShapeDtypeStruct((B, 1, H, D), jnp.float32),
        scratch_shapes=[
            pltpu.VMEM((B, H, CW), jnp.float32),
            pltpu.VMEM((B, H, CW), jnp.float32),
            pltpu.SemaphoreType.DMA,
            pltpu.SemaphoreType.DMA,
        ],
        compiler_params=pltpu.CompilerParams(collective_id=0),
    )(Q, K, V)


# baseline (device time: 377270 ns/iter reference)
import jax
import jax.numpy as jnp
from jax import lax
from jax.experimental import pallas as pl
from jax.experimental.pallas import tpu as pltpu

B = 16
H = 16
D = 64
HB = 8
SCALE = D ** -0.5
CW = D + 2


def kernel(Q, K, V):
    kv_local = K.shape[1]

    def body(q_ref, k_ref, v_ref, out_ref, send_ref, recv_ref,
             send_sem, recv_sem):
        b = pl.program_id(0)
        hg = pl.program_id(1)
        nb = pl.num_programs(0)
        nhg = pl.num_programs(1)

        for h0 in range(HB):
            h = hg * HB + h0
            k2 = k_ref[0, :, h0, :]
            v2 = v_ref[0, :, h0, :]
            q2 = q_ref[b, :, h, :]
            s = lax.dot_general(
                q2, k2, (((1,), (1,)), ((), ())),
                preferred_element_type=jnp.float32,
            ) * SCALE
            m = jnp.max(s, axis=1, keepdims=True)
            p = jnp.exp(s - m)
            l = jnp.sum(p, axis=1, keepdims=True)
            o = lax.dot_general(
                p, v2, (((1,), (0,)), ((), ())),
                preferred_element_type=jnp.float32,
            )
            send_ref[b, pl.ds(h, 1), pl.ds(0, D)] = o
            send_ref[b, pl.ds(h, 1), pl.ds(D, 1)] = m
            send_ref[b, pl.ds(h, 1), pl.ds(D + 1, 1)] = l

        @pl.when((b == nb - 1) & (hg == nhg - 1))
        def _():
            my_x = lax.axis_index("x")
            my_y = lax.axis_index("y")
            my_z = lax.axis_index("z")
            nbr = (1 - my_x, my_y, my_z)

            barrier = pltpu.get_barrier_semaphore()
            pl.semaphore_signal(
                barrier, inc=1, device_id=nbr,
                device_id_type=pl.DeviceIdType.MESH,
            )
            pl.semaphore_wait(barrier, 1)

            rdma = pltpu.make_async_remote_copy(
                src_ref=send_ref,
                dst_ref=recv_ref,
                send_sem=send_sem,
                recv_sem=recv_sem,
                device_id=nbr,
                device_id_type=pl.DeviceIdType.MESH,
            )
            rdma.start()
            rdma.wait()

            o_a = send_ref[:, :, 0:D]
            m_a = send_ref[:, :, D:D + 1]
            l_a = send_ref[:, :, D + 1:D + 2]
            o_b = recv_ref[:, :, 0:D]
            m_b = recv_ref[:, :, D:D + 1]
            l_b = recv_ref[:, :, D + 1:D + 2]
            m_n = jnp.maximum(m_a, m_b)
            alpha = jnp.exp(m_a - m_n)
            beta = jnp.exp(m_b - m_n)
            l_n = l_a * alpha + l_b * beta
            out_ref[:, 0, :, :] = (o_a * alpha + o_b * beta) / l_n

    return pl.pallas_call(
        body,
        grid=(B, H // HB),
        in_specs=[
            pl.BlockSpec((B, 1, H, D), lambda b, hg: (0, 0, 0, 0)),
            pl.BlockSpec((1, kv_local, HB, D), lambda b, hg: (b, 0, hg, 0)),
            pl.BlockSpec((1, kv_local, HB, D), lambda b, hg: (b, 0, hg, 0)),
        ],
        out_specs=pl.BlockSpec((B, 1, H, D), lambda b, hg: (0, 0, 0, 0)),
        out_shape=jax.ShapeDtypeStruct((B, 1, H, D), jnp.float32),
        scratch_shapes=[
            pltpu.VMEM((B, H, CW), jnp.float32),
            pltpu.VMEM((B, H, CW), jnp.float32),
            pltpu.SemaphoreType.DMA,
            pltpu.SemaphoreType.DMA,
        ],
        compiler_params=pltpu.CompilerParams(collective_id=0),
    )(Q, K, V)
